# baseline (speedup 1.0000x reference)
import functools

import jax
import jax.numpy as jnp
from jax import lax
from jax.experimental import pallas as pl
from jax.experimental.pallas import tpu as pltpu
from jax.experimental.pallas import tpu_sc as plsc

NC = 2
NS = 16
CHUNK = 64


def _ceil_to(a, m):
    return (a + m - 1) // m * m


def _row_block(n):
    for cand in (2000, 1000, 500, 400, 250, 200, 125, 100, 50, 40, 25, 20,
                 16, 10, 8, 5, 4, 2, 1):
        if n % cand == 0:
            return cand


def _pack_bf16(xs):
    half = xs.shape[1] // 2
    xb = xs.astype(jnp.bfloat16)
    lo = jax.lax.bitcast_convert_type(xb[:, :half], jnp.uint16).astype(jnp.uint32)
    hi = jax.lax.bitcast_convert_type(xb[:, half:], jnp.uint16).astype(jnp.uint32)
    return jax.lax.bitcast_convert_type((hi << 16) | lo, jnp.int32)


def _proj_kernel(h, w, b):
    n, k = h.shape
    blk = _row_block(n)
    def body(h_ref, w_ref, b_ref, o_ref):
        acc = jnp.dot(h_ref[...], w_ref[...], preferred_element_type=jnp.float32)
        o_ref[...] = _pack_bf16(jnp.maximum(acc + b_ref[...], 0.0))
    return pl.pallas_call(
        body,
        grid=(n // blk,),
        in_specs=[
            pl.BlockSpec((blk, k), lambda i: (i, 0)),
            pl.BlockSpec((k, k), lambda i: (0, 0)),
            pl.BlockSpec((1, k), lambda i: (0, 0)),
        ],
        out_specs=pl.BlockSpec((blk, k // 2), lambda i: (i, 0)),
        out_shape=jax.ShapeDtypeStruct((n, k // 2), jnp.int32),
    )(h, w, b)


def _tail_split(p0, p1, h, wl, bl, wr, relu):
    n = h.shape[0]
    k = p0.shape[1]
    o = wl.shape[1]
    blk = _row_block(n)
    def body(p0_ref, p1_ref, h_ref, wl_ref, bl_ref, wr_ref, o_ref):
        agg = p0_ref[...] + p1_ref[...]
        acc = jnp.dot(agg, wl_ref[...], preferred_element_type=jnp.float32)
        acc = acc + bl_ref[...]
        acc = acc + jnp.dot(h_ref[...], wr_ref[...], preferred_element_type=jnp.float32)
        if relu:
            acc = jnp.maximum(acc, 0.0)
        o_ref[...] = acc
    return pl.pallas_call(
        body,
        grid=(n // blk,),
        in_specs=[
            pl.BlockSpec((blk, k), lambda i: (i, 0)),
            pl.BlockSpec((blk, k), lambda i: (i, 0)),
            pl.BlockSpec((blk, k), lambda i: (i, 0)),
            pl.BlockSpec((k, o), lambda i: (0, 0)),
            pl.BlockSpec((1, o), lambda i: (0, 0)),
            pl.BlockSpec((k, o), lambda i: (0, 0)),
        ],
        out_specs=pl.BlockSpec((blk, o), lambda i: (i, 0)),
        out_shape=jax.ShapeDtypeStruct((n, o), jnp.float32),
    )(p0, p1, h, wl, bl, wr)


def _tail_kernel(parts, h, wl, bl, wr, relu, next_proj=None):
    n_acc2, k = parts.shape
    n_acc = n_acc2 // NC
    n = h.shape[0]
    o = wl.shape[1]
    blk = _row_block(n)
    if n_acc % blk != 0:
        out = _tail_split(parts[:n_acc], parts[n_acc:], h, wl, bl, wr, relu)
        if next_proj is None:
            return out
        return out, _proj_kernel(out, next_proj[0], next_proj[1].reshape(1, -1))

    def body(p0_ref, p1_ref, h_ref, wl_ref, bl_ref, wr_ref, *rest):
        if next_proj is not None:
            pw_ref, pb_ref, o_ref, xs_ref = rest
        else:
            (o_ref,) = rest
        agg = p0_ref[...] + p1_ref[...]
        acc = jnp.dot(agg, wl_ref[...], preferred_element_type=jnp.float32)
        acc = acc + bl_ref[...]
        acc = acc + jnp.dot(h_ref[...], wr_ref[...], preferred_element_type=jnp.float32)
        if relu:
            acc = jnp.maximum(acc, 0.0)
        o_ref[...] = acc
        if next_proj is not None:
            xs = jnp.dot(acc, pw_ref[...], preferred_element_type=jnp.float32)
            xs_ref[...] = _pack_bf16(jnp.maximum(xs + pb_ref[...], 0.0))

    nb = n_acc // blk
    in_specs = [
        pl.BlockSpec((blk, k), lambda i: (i, 0)),
        pl.BlockSpec((blk, k), lambda i, nb=nb: (nb + i, 0)),
        pl.BlockSpec((blk, k), lambda i: (i, 0)),
        pl.BlockSpec((k, o), lambda i: (0, 0)),
        pl.BlockSpec((1, o), lambda i: (0, 0)),
        pl.BlockSpec((k, o), lambda i: (0, 0)),
    ]
    args = [parts, parts, h, wl, bl, wr]
    out_specs = pl.BlockSpec((blk, o), lambda i: (i, 0))
    out_shape = jax.ShapeDtypeStruct((n, o), jnp.float32)
    if next_proj is not None:
        p_w, p_b = next_proj
        in_specs += [
            pl.BlockSpec((o, o), lambda i: (0, 0)),
            pl.BlockSpec((1, o), lambda i: (0, 0)),
        ]
        args += [p_w, p_b.reshape(1, -1)]
        out_specs = [out_specs, pl.BlockSpec((blk, o // 2), lambda i: (i, 0))]
        out_shape = [out_shape, jax.ShapeDtypeStruct((n, o // 2), jnp.int32)]
    return pl.pallas_call(
        body,
        grid=(n // blk,),
        in_specs=in_specs,
        out_specs=out_specs,
        out_shape=out_shape,
    )(*args)


def _segment_partials(xsrc_packed, d, src2, dst2, zeros_hbm, n_acc, k0_total):
    dp = d // 2
    k_ch = src2.shape[0]
    k1_total = k_ch - k0_total
    rpt = n_acc // NS
    ib = 40
    nbuf = 2 if d >= 128 else 4
    zfull = rpt - rpt % CHUNK
    ztail = rpt % CHUNK
    mesh = plsc.VectorSubcoreMesh(core_axis_name="c", subcore_axis_name="s")

    @functools.partial(
        pl.kernel,
        out_type=jax.ShapeDtypeStruct((NC * n_acc, d), jnp.float32),
        mesh=mesh,
        scratch_types=[
            pltpu.VMEM((ib, CHUNK), jnp.int32),
            pltpu.VMEM((ib, CHUNK), jnp.int32),
            [pltpu.VMEM((CHUNK, dp), jnp.int32) for _ in range(nbuf)],
            [pltpu.VMEM((CHUNK, d), jnp.float32) for _ in range(nbuf)],
            pltpu.VMEM_SHARED((n_acc, d), jnp.float32),
            [pltpu.SemaphoreType.DMA for _ in range(nbuf)],
            [pltpu.SemaphoreType.DMA for _ in range(nbuf)],
        ],
        compiler_params=pltpu.CompilerParams(use_tc_tiling_on_sc=False,
                                             needs_layout_passes=False),
    )
    def seg_kernel(xsrc_hbm, src_hbm, dst_hbm, z_hbm, out_hbm,
                   src_v, dst_v, gbufs, fbufs, agg_sh, gsems, ssems):
        c = lax.axis_index("c")
        s = lax.axis_index("s")
        base = s * rpt

        def unpack_chunk(gbuf, fbuf):
            @pl.loop(0, CHUNK, unroll=4)
            def _rows(r):
                for g in range(dp // 16):
                    v = gbuf[r, pl.ds(16 * g, 16)]
                    fbuf[r, pl.ds(16 * g, 16)] = plsc.bitcast(
                        jnp.left_shift(v, 16), jnp.float32)
                    fbuf[r, pl.ds(16 * g + dp, 16)] = plsc.bitcast(
                        jnp.bitwise_and(v, jnp.int32(-65536)), jnp.float32)

        pltpu.sync_copy(z_hbm, fbufs[0])

        @pl.loop(0, zfull, step=CHUNK)
        def _zero(r):
            pltpu.sync_copy(fbufs[0], agg_sh.at[pl.ds(base + r, CHUNK)])

        if ztail:
            pltpu.sync_copy(fbufs[0].at[pl.ds(0, ztail)],
                            agg_sh.at[pl.ds(base + zfull, ztail)])

        plsc.subcore_barrier()

        def run_edges(kc, t0):
            for b0 in range(0, kc, ib):
                nbl = min(ib, kc - b0)
                pltpu.sync_copy(src_hbm.at[pl.ds(t0 + b0, nbl)],
                                src_v.at[pl.ds(0, nbl)])
                pltpu.sync_copy(dst_hbm.at[pl.ds(t0 + b0, nbl)],
                                dst_v.at[pl.ds(0, nbl)])
                nb = min(nbuf, nbl)
                for i in range(nb):
                    pltpu.async_copy(xsrc_hbm.at[src_v.at[i]], gbufs[i], gsems[i])
                full = nbl - (nbl % nb)

                @pl.loop(0, full, step=nb)
                def _edges(j):
                    for b in range(nb):
                        pltpu.make_async_copy(
                            xsrc_hbm.at[src_v.at[j + b]], gbufs[b], gsems[b]).wait()

                        @pl.when(j + b >= nb)
                        def _wsc(b=b):
                            pltpu.make_async_copy(
                                fbufs[b], agg_sh.at[dst_v.at[0]], ssems[b]).wait()

                        unpack_chunk(gbufs[b], fbufs[b])

                        @pl.when(j + b + nb < nbl)
                        def _refill(b=b):
                            pltpu.async_copy(xsrc_hbm.at[src_v.at[j + b + nb]],
                                             gbufs[b], gsems[b])

                        pltpu.async_copy(fbufs[b], agg_sh.at[dst_v.at[j + b]],
                                         ssems[b], add=True)

                for r in range(full, nbl):
                    b = r % nb
                    pltpu.make_async_copy(
                        xsrc_hbm.at[src_v.at[r]], gbufs[b], gsems[b]).wait()
                    if r >= nb:
                        pltpu.make_async_copy(
                            fbufs[b], agg_sh.at[dst_v.at[0]], ssems[b]).wait()
                    unpack_chunk(gbufs[b], fbufs[b])
                    pltpu.async_copy(fbufs[b], agg_sh.at[dst_v.at[r]],
                                     ssems[b], add=True)

                for b in range(min(nb, nbl)):
                    pltpu.make_async_copy(
                        fbufs[b], agg_sh.at[dst_v.at[0]], ssems[b]).wait()

        def core_edges(k_total, core_base):
            q, r = divmod(k_total, NS)
            t0 = core_base + s * q + jnp.minimum(s, r)
            if r > 0:
                @pl.when(s < r)
                def _extra():
                    run_edges(q + 1, t0)

                if q > 0:
                    @pl.when(s >= r)
                    def _plain():
                        run_edges(q, t0)
            elif q > 0:
                run_edges(q, t0)

        @pl.when(c == 0)
        def _core0():
            core_edges(k0_total, 0)

        @pl.when(c == 1)
        def _core1():
            core_edges(k1_total, k0_total)

        plsc.subcore_barrier()

        @pl.loop(0, zfull, step=CHUNK)
        def _out(r):
            pltpu.sync_copy(agg_sh.at[pl.ds(base + r, CHUNK)], fbufs[0])
            pltpu.sync_copy(fbufs[0], out_hbm.at[pl.ds(c * n_acc + base + r, CHUNK)])

        if ztail:
            pltpu.sync_copy(agg_sh.at[pl.ds(base + zfull, ztail)],
                            fbufs[0].at[pl.ds(0, ztail)])
            pltpu.sync_copy(fbufs[0].at[pl.ds(0, ztail)],
                            out_hbm.at[pl.ds(c * n_acc + base + zfull, ztail)])

    return seg_kernel(xsrc_packed, src2, dst2, zeros_hbm)


def kernel(x, edge_index,
           enc1_proj_W, enc1_proj_b, enc1_lin_l_W, enc1_lin_l_b, enc1_lin_r_W,
           enc2_proj_W, enc2_proj_b, enc2_lin_l_W, enc2_lin_l_b, enc2_lin_r_W,
           dec1_proj_W, dec1_proj_b, dec1_lin_l_W, dec1_lin_l_b, dec1_lin_r_W,
           dec2_proj_W, dec2_proj_b, dec2_lin_l_W, dec2_lin_l_b, dec2_lin_r_W):
    n, d_in = x.shape
    e = edge_index.shape[1]
    n_acc = _ceil_to(n, NS)

    pad = _ceil_to(e, CHUNK) - e
    if pad:
        n_acc = _ceil_to(n + 1, NS)
        src_flat = jnp.concatenate([edge_index[0], jnp.zeros((pad,), jnp.int32)])
        dst_flat = jnp.concatenate([edge_index[1], jnp.full((pad,), n, jnp.int32)])
    else:
        src_flat = edge_index[0]
        dst_flat = edge_index[1]
    k_ch = (e + pad) // CHUNK
    src2 = src_flat.reshape(k_ch, CHUNK)
    dst2 = dst_flat.reshape(k_ch, CHUNK)

    frac0 = {128: 0.72, 64: 0.64, 32: 0.57}

    def seg(xsrc_packed, d):
        zeros_hbm = jnp.zeros((CHUNK, d), jnp.float32)
        k0_total = min(k_ch - 1, max(1, round(k_ch * frac0.get(d, 0.5))))
        return _segment_partials(xsrc_packed, d, src2, dst2, zeros_hbm,
                                 n_acc, k0_total)

    xsrc = _proj_kernel(x, enc1_proj_W, enc1_proj_b.reshape(1, -1))
    parts = seg(xsrc, d_in)
    h, xsrc = _tail_kernel(parts, x, enc1_lin_l_W, enc1_lin_l_b.reshape(1, -1),
                           enc1_lin_r_W, True, (enc2_proj_W, enc2_proj_b))
    parts = seg(xsrc, h.shape[1])
    z, xsrc = _tail_kernel(parts, h, enc2_lin_l_W, enc2_lin_l_b.reshape(1, -1),
                           enc2_lin_r_W, False, (dec1_proj_W, dec1_proj_b))
    parts = seg(xsrc, z.shape[1])
    h2, xsrc = _tail_kernel(parts, z, dec1_lin_l_W, dec1_lin_l_b.reshape(1, -1),
                            dec1_lin_r_W, True, (dec2_proj_W, dec2_proj_b))
    parts = seg(xsrc, h2.shape[1])
    xr = _tail_kernel(parts, h2, dec2_lin_l_W, dec2_lin_l_b.reshape(1, -1),
                      dec2_lin_r_W, False)
    return (xr, z)


# speedup vs baseline: 1.7665x; 1.7665x over previous
import functools

import jax
import jax.numpy as jnp
from jax import lax
from jax.experimental import pallas as pl
from jax.experimental.pallas import tpu as pltpu
from jax.experimental.pallas import tpu_sc as plsc

NC = 2
NS = 16
CHUNK = 128


def _ceil_to(a, m):
    return (a + m - 1) // m * m


def _row_block(n):
    for cand in (2000, 1000, 500, 400, 250, 200, 125, 100, 50, 40, 25, 20,
                 16, 10, 8, 5, 4, 2, 1):
        if n % cand == 0:
            return cand


def _proj_kernel(h, w, b):
    n, k = h.shape
    blk = _row_block(n)
    def body(h_ref, w_ref, b_ref, o_ref):
        acc = jnp.dot(h_ref[...], w_ref[...], preferred_element_type=jnp.float32)
        o_ref[...] = jnp.maximum(acc + b_ref[...], 0.0)
    return pl.pallas_call(
        body,
        grid=(n // blk,),
        in_specs=[
            pl.BlockSpec((blk, k), lambda i: (i, 0)),
            pl.BlockSpec((k, k), lambda i: (0, 0)),
            pl.BlockSpec((1, k), lambda i: (0, 0)),
        ],
        out_specs=pl.BlockSpec((blk, k), lambda i: (i, 0)),
        out_shape=jax.ShapeDtypeStruct((n, k), jnp.float32),
    )(h, w, b)


def _tail_split(p0, p1, h, wl, bl, wr, relu):
    n = h.shape[0]
    k = p0.shape[1]
    o = wl.shape[1]
    blk = _row_block(n)
    def body(p0_ref, p1_ref, h_ref, wl_ref, bl_ref, wr_ref, o_ref):
        agg = p0_ref[...] + p1_ref[...]
        acc = jnp.dot(agg, wl_ref[...], preferred_element_type=jnp.float32)
        acc = acc + bl_ref[...]
        acc = acc + jnp.dot(h_ref[...], wr_ref[...], preferred_element_type=jnp.float32)
        if relu:
            acc = jnp.maximum(acc, 0.0)
        o_ref[...] = acc
    return pl.pallas_call(
        body,
        grid=(n // blk,),
        in_specs=[
            pl.BlockSpec((blk, k), lambda i: (i, 0)),
            pl.BlockSpec((blk, k), lambda i: (i, 0)),
            pl.BlockSpec((blk, k), lambda i: (i, 0)),
            pl.BlockSpec((k, o), lambda i: (0, 0)),
            pl.BlockSpec((1, o), lambda i: (0, 0)),
            pl.BlockSpec((k, o), lambda i: (0, 0)),
        ],
        out_specs=pl.BlockSpec((blk, o), lambda i: (i, 0)),
        out_shape=jax.ShapeDtypeStruct((n, o), jnp.float32),
    )(p0, p1, h, wl, bl, wr)


def _tail_kernel(parts, h, wl, bl, wr, relu, next_proj=None):
    n_acc2, k = parts.shape
    n_acc = n_acc2 // NC
    n = h.shape[0]
    o = wl.shape[1]
    blk = _row_block(n)
    if n_acc % blk != 0:
        out = _tail_split(parts[:n_acc], parts[n_acc:], h, wl, bl, wr, relu)
        if next_proj is None:
            return out
        return out, _proj_kernel(out, next_proj[0], next_proj[1])

    def body(p0_ref, p1_ref, h_ref, wl_ref, bl_ref, wr_ref, *rest):
        if next_proj is not None:
            pw_ref, pb_ref, o_ref, xs_ref = rest
        else:
            (o_ref,) = rest
        agg = p0_ref[...] + p1_ref[...]
        acc = jnp.dot(agg, wl_ref[...], preferred_element_type=jnp.float32)
        acc = acc + bl_ref[...]
        acc = acc + jnp.dot(h_ref[...], wr_ref[...], preferred_element_type=jnp.float32)
        if relu:
            acc = jnp.maximum(acc, 0.0)
        o_ref[...] = acc
        if next_proj is not None:
            xs = jnp.dot(acc, pw_ref[...], preferred_element_type=jnp.float32)
            xs_ref[...] = jnp.maximum(xs + pb_ref[...], 0.0)

    nb = n_acc // blk
    in_specs = [
        pl.BlockSpec((blk, k), lambda i: (i, 0)),
        pl.BlockSpec((blk, k), lambda i, nb=nb: (nb + i, 0)),
        pl.BlockSpec((blk, k), lambda i: (i, 0)),
        pl.BlockSpec((k, o), lambda i: (0, 0)),
        pl.BlockSpec((1, o), lambda i: (0, 0)),
        pl.BlockSpec((k, o), lambda i: (0, 0)),
    ]
    args = [parts, parts, h, wl, bl, wr]
    out_specs = pl.BlockSpec((blk, o), lambda i: (i, 0))
    out_shape = jax.ShapeDtypeStruct((n, o), jnp.float32)
    if next_proj is not None:
        p_w, p_b = next_proj
        in_specs += [
            pl.BlockSpec((o, o), lambda i: (0, 0)),
            pl.BlockSpec((1, o), lambda i: (0, 0)),
        ]
        args += [p_w, p_b.reshape(1, -1)]
        out_specs = [out_specs, pl.BlockSpec((blk, o), lambda i: (i, 0))]
        out_shape = [out_shape, jax.ShapeDtypeStruct((n, o), jnp.float32)]
    return pl.pallas_call(
        body,
        grid=(n // blk,),
        in_specs=in_specs,
        out_specs=out_specs,
        out_shape=out_shape,
    )(*args)


def _segment_partials(xsrc, src2, dst2, zeros_hbm, n_acc, k0_total):
    d = xsrc.shape[1]
    k_ch = src2.shape[0]
    k1_total = k_ch - k0_total
    rpt = n_acc // NS
    ib = 40
    nbuf = 2 if d >= 128 else 4
    zfull = rpt - rpt % CHUNK
    ztail = rpt % CHUNK
    mesh = plsc.VectorSubcoreMesh(core_axis_name="c", subcore_axis_name="s")

    @functools.partial(
        pl.kernel,
        out_type=jax.ShapeDtypeStruct((NC * n_acc, d), jnp.float32),
        mesh=mesh,
        scratch_types=[
            pltpu.VMEM((ib, CHUNK), jnp.int32),
            pltpu.VMEM((ib, CHUNK), jnp.int32),
            [pltpu.VMEM((CHUNK, d), jnp.float32) for _ in range(nbuf)],
            pltpu.VMEM_SHARED((n_acc, d), jnp.float32),
            [pltpu.SemaphoreType.DMA for _ in range(nbuf)],
        ],
        compiler_params=pltpu.CompilerParams(use_tc_tiling_on_sc=False),
    )
    def seg_kernel(xsrc_hbm, src_hbm, dst_hbm, z_hbm, out_hbm,
                   src_v, dst_v, rows, agg_sh, sems):
        c = lax.axis_index("c")
        s = lax.axis_index("s")
        base = s * rpt

        pltpu.sync_copy(z_hbm, rows[0])

        @pl.loop(0, zfull, step=CHUNK)
        def _zero(r):
            pltpu.sync_copy(rows[0], agg_sh.at[pl.ds(base + r, CHUNK)])

        if ztail:
            pltpu.sync_copy(rows[0].at[pl.ds(0, ztail)],
                            agg_sh.at[pl.ds(base + zfull, ztail)])

        plsc.subcore_barrier()

        def run_edges(kc, t0):
            for b0 in range(0, kc, ib):
                nbl = min(ib, kc - b0)
                pltpu.sync_copy(src_hbm.at[pl.ds(t0 + b0, nbl)],
                                src_v.at[pl.ds(0, nbl)])
                pltpu.sync_copy(dst_hbm.at[pl.ds(t0 + b0, nbl)],
                                dst_v.at[pl.ds(0, nbl)])
                nb = min(nbuf, nbl)
                for i in range(nb):
                    pltpu.async_copy(xsrc_hbm.at[src_v.at[i]], rows[i], sems[i])
                full = nbl - (nbl % nb)

                @pl.loop(0, full, step=nb)
                def _edges(j):
                    for b in range(nb):
                        pltpu.make_async_copy(
                            xsrc_hbm.at[src_v.at[j + b]], rows[b], sems[b]).wait()
                        pltpu.sync_copy(rows[b], agg_sh.at[dst_v.at[j + b]],
                                        add=True)

                        @pl.when(j + b + nb < nbl)
                        def _refill(b=b):
                            pltpu.async_copy(xsrc_hbm.at[src_v.at[j + b + nb]],
                                             rows[b], sems[b])

                for r in range(full, nbl):
                    b = r % nb
                    pltpu.make_async_copy(
                        xsrc_hbm.at[src_v.at[r]], rows[b], sems[b]).wait()
                    pltpu.sync_copy(rows[b], agg_sh.at[dst_v.at[r]], add=True)

        def core_edges(k_total, core_base):
            q, r = divmod(k_total, NS)
            t0 = core_base + s * q + jnp.minimum(s, r)
            if r > 0:
                @pl.when(s < r)
                def _extra():
                    run_edges(q + 1, t0)

                if q > 0:
                    @pl.when(s >= r)
                    def _plain():
                        run_edges(q, t0)
            elif q > 0:
                run_edges(q, t0)

        @pl.when(c == 0)
        def _core0():
            core_edges(k0_total, 0)

        @pl.when(c == 1)
        def _core1():
            core_edges(k1_total, k0_total)

        plsc.subcore_barrier()

        pltpu.sync_copy(agg_sh.at[pl.ds(base, rpt)],
                        out_hbm.at[pl.ds(c * n_acc + base, rpt)])

    return seg_kernel(xsrc, src2, dst2, zeros_hbm)


def kernel(x, edge_index,
           enc1_proj_W, enc1_proj_b, enc1_lin_l_W, enc1_lin_l_b, enc1_lin_r_W,
           enc2_proj_W, enc2_proj_b, enc2_lin_l_W, enc2_lin_l_b, enc2_lin_r_W,
           dec1_proj_W, dec1_proj_b, dec1_lin_l_W, dec1_lin_l_b, dec1_lin_r_W,
           dec2_proj_W, dec2_proj_b, dec2_lin_l_W, dec2_lin_l_b, dec2_lin_r_W):
    n, d_in = x.shape
    e = edge_index.shape[1]
    n_acc = _ceil_to(n, NS)

    pad = _ceil_to(e, CHUNK) - e
    if pad:
        n_acc = _ceil_to(n + 1, NS)
        src_flat = jnp.concatenate([edge_index[0], jnp.zeros((pad,), jnp.int32)])
        dst_flat = jnp.concatenate([edge_index[1], jnp.full((pad,), n, jnp.int32)])
    else:
        src_flat = edge_index[0]
        dst_flat = edge_index[1]
    k_ch = (e + pad) // CHUNK
    src2 = src_flat.reshape(k_ch, CHUNK)
    dst2 = dst_flat.reshape(k_ch, CHUNK)

    frac0 = {128: 0.72, 64: 0.64, 32: 0.57}

    def seg(xsrc, k0_total):
        d = xsrc.shape[1]
        zeros_hbm = jnp.zeros((CHUNK, d), jnp.float32)
        return _segment_partials(xsrc, src2, dst2, zeros_hbm, n_acc, k0_total)

    def k0_for(d):
        return min(k_ch - 1, max(1, round(k_ch * frac0.get(d, 0.5))))

    xsrc = _proj_kernel(x, enc1_proj_W, enc1_proj_b.reshape(1, -1))
    parts = seg(xsrc, k0_for(d_in))
    h, xsrc = _tail_kernel(parts, x, enc1_lin_l_W, enc1_lin_l_b.reshape(1, -1),
                           enc1_lin_r_W, True, (enc2_proj_W, enc2_proj_b))
    parts = seg(xsrc, k0_for(h.shape[1]))
    z, xsrc = _tail_kernel(parts, h, enc2_lin_l_W, enc2_lin_l_b.reshape(1, -1),
                           enc2_lin_r_W, False, (dec1_proj_W, dec1_proj_b))
    parts = seg(xsrc, k0_for(z.shape[1]))
    h2, xsrc = _tail_kernel(parts, z, dec1_lin_l_W, dec1_lin_l_b.reshape(1, -1),
                            dec1_lin_r_W, True, (dec2_proj_W, dec2_proj_b))
    parts = seg(xsrc, k0_for(h2.shape[1]))
    xr = _tail_kernel(parts, h2, dec2_lin_l_W, dec2_lin_l_b.reshape(1, -1),
                      dec2_lin_r_W, False)
    return (xr, z)


# speedup vs baseline: 1.7739x; 1.0042x over previous
import functools

import jax
import jax.numpy as jnp
from jax import lax
from jax.experimental import pallas as pl
from jax.experimental.pallas import tpu as pltpu
from jax.experimental.pallas import tpu_sc as plsc

NC = 2
NS = 16
CHUNK = 128


def _ceil_to(a, m):
    return (a + m - 1) // m * m


def _row_block(n):
    for cand in (2000, 1000, 500, 400, 250, 200, 125, 100, 50, 40, 25, 20,
                 16, 10, 8, 5, 4, 2, 1):
        if n % cand == 0:
            return cand


def _proj_kernel(h, w, b):
    n, k = h.shape
    blk = _row_block(n)
    def body(h_ref, w_ref, b_ref, o_ref):
        acc = jnp.dot(h_ref[...], w_ref[...], preferred_element_type=jnp.float32)
        o_ref[...] = jnp.maximum(acc + b_ref[...], 0.0)
    return pl.pallas_call(
        body,
        grid=(n // blk,),
        in_specs=[
            pl.BlockSpec((blk, k), lambda i: (i, 0)),
            pl.BlockSpec((k, k), lambda i: (0, 0)),
            pl.BlockSpec((1, k), lambda i: (0, 0)),
        ],
        out_specs=pl.BlockSpec((blk, k), lambda i: (i, 0)),
        out_shape=jax.ShapeDtypeStruct((n, k), jnp.float32),
    )(h, w, b)


def _tail_split(p0, p1, h, wl, bl, wr, relu):
    n = h.shape[0]
    k = p0.shape[1]
    o = wl.shape[1]
    blk = _row_block(n)
    def body(p0_ref, p1_ref, h_ref, wl_ref, bl_ref, wr_ref, o_ref):
        agg = p0_ref[...] + p1_ref[...]
        acc = jnp.dot(agg, wl_ref[...], preferred_element_type=jnp.float32)
        acc = acc + bl_ref[...]
        acc = acc + jnp.dot(h_ref[...], wr_ref[...], preferred_element_type=jnp.float32)
        if relu:
            acc = jnp.maximum(acc, 0.0)
        o_ref[...] = acc
    return pl.pallas_call(
        body,
        grid=(n // blk,),
        in_specs=[
            pl.BlockSpec((blk, k), lambda i: (i, 0)),
            pl.BlockSpec((blk, k), lambda i: (i, 0)),
            pl.BlockSpec((blk, k), lambda i: (i, 0)),
            pl.BlockSpec((k, o), lambda i: (0, 0)),
            pl.BlockSpec((1, o), lambda i: (0, 0)),
            pl.BlockSpec((k, o), lambda i: (0, 0)),
        ],
        out_specs=pl.BlockSpec((blk, o), lambda i: (i, 0)),
        out_shape=jax.ShapeDtypeStruct((n, o), jnp.float32),
    )(p0, p1, h, wl, bl, wr)


def _tail_kernel(parts, h, wl, bl, wr, relu, next_proj=None):
    n_acc2, k = parts.shape
    n_acc = n_acc2 // NC
    n = h.shape[0]
    o = wl.shape[1]
    blk = _row_block(n)
    if n_acc % blk != 0:
        out = _tail_split(parts[:n_acc], parts[n_acc:], h, wl, bl, wr, relu)
        if next_proj is None:
            return out
        return out, _proj_kernel(out, next_proj[0], next_proj[1])

    def body(p0_ref, p1_ref, h_ref, wl_ref, bl_ref, wr_ref, *rest):
        if next_proj is not None:
            pw_ref, pb_ref, o_ref, xs_ref = rest
        else:
            (o_ref,) = rest
        agg = p0_ref[...] + p1_ref[...]
        acc = jnp.dot(agg, wl_ref[...], preferred_element_type=jnp.float32)
        acc = acc + bl_ref[...]
        acc = acc + jnp.dot(h_ref[...], wr_ref[...], preferred_element_type=jnp.float32)
        if relu:
            acc = jnp.maximum(acc, 0.0)
        o_ref[...] = acc
        if next_proj is not None:
            xs = jnp.dot(acc, pw_ref[...], preferred_element_type=jnp.float32)
            xs_ref[...] = jnp.maximum(xs + pb_ref[...], 0.0)

    nb = n_acc // blk
    in_specs = [
        pl.BlockSpec((blk, k), lambda i: (i, 0)),
        pl.BlockSpec((blk, k), lambda i, nb=nb: (nb + i, 0)),
        pl.BlockSpec((blk, k), lambda i: (i, 0)),
        pl.BlockSpec((k, o), lambda i: (0, 0)),
        pl.BlockSpec((1, o), lambda i: (0, 0)),
        pl.BlockSpec((k, o), lambda i: (0, 0)),
    ]
    args = [parts, parts, h, wl, bl, wr]
    out_specs = pl.BlockSpec((blk, o), lambda i: (i, 0))
    out_shape = jax.ShapeDtypeStruct((n, o), jnp.float32)
    if next_proj is not None:
        p_w, p_b = next_proj
        in_specs += [
            pl.BlockSpec((o, o), lambda i: (0, 0)),
            pl.BlockSpec((1, o), lambda i: (0, 0)),
        ]
        args += [p_w, p_b.reshape(1, -1)]
        out_specs = [out_specs, pl.BlockSpec((blk, o), lambda i: (i, 0))]
        out_shape = [out_shape, jax.ShapeDtypeStruct((n, o), jnp.float32)]
    return pl.pallas_call(
        body,
        grid=(n // blk,),
        in_specs=in_specs,
        out_specs=out_specs,
        out_shape=out_shape,
    )(*args)


def _segment_partials(xsrc, src2, dst2, zeros_hbm, n_acc, k0_total):
    d = xsrc.shape[1]
    k_ch = src2.shape[0]
    k1_total = k_ch - k0_total
    rpt = n_acc // NS
    ib = 40
    nbuf = 2 if d >= 128 else 4
    zfull = rpt - rpt % CHUNK
    ztail = rpt % CHUNK
    mesh = plsc.VectorSubcoreMesh(core_axis_name="c", subcore_axis_name="s")

    @functools.partial(
        pl.kernel,
        out_type=jax.ShapeDtypeStruct((NC * n_acc, d), jnp.float32),
        mesh=mesh,
        scratch_types=[
            pltpu.VMEM((ib, CHUNK), jnp.int32),
            pltpu.VMEM((ib, CHUNK), jnp.int32),
            [pltpu.VMEM((CHUNK, d), jnp.float32) for _ in range(nbuf)],
            pltpu.VMEM_SHARED((n_acc, d), jnp.float32),
            [pltpu.SemaphoreType.DMA for _ in range(nbuf)],
        ],
        compiler_params=pltpu.CompilerParams(use_tc_tiling_on_sc=False),
    )
    def seg_kernel(xsrc_hbm, src_hbm, dst_hbm, z_hbm, out_hbm,
                   src_v, dst_v, rows, agg_sh, sems):
        c = lax.axis_index("c")
        s = lax.axis_index("s")
        base = s * rpt

        pltpu.sync_copy(z_hbm, rows[0])
        zcopies = []
        for i, r in enumerate(range(0, zfull, CHUNK)):
            zcopies.append((rows[0], agg_sh.at[pl.ds(base + r, CHUNK)],
                            sems[i % nbuf]))
        if ztail:
            zcopies.append((rows[0].at[pl.ds(0, ztail)],
                            agg_sh.at[pl.ds(base + zfull, ztail)],
                            sems[len(zcopies) % nbuf]))
        for srcr, dstr, sm in zcopies:
            pltpu.async_copy(srcr, dstr, sm)
        for srcr, dstr, sm in zcopies:
            pltpu.make_async_copy(srcr, dstr, sm).wait()

        plsc.subcore_barrier()

        def run_edges(kc, t0):
            for b0 in range(0, kc, ib):
                nbl = min(ib, kc - b0)
                pltpu.sync_copy(src_hbm.at[pl.ds(t0 + b0, nbl)],
                                src_v.at[pl.ds(0, nbl)])
                pltpu.sync_copy(dst_hbm.at[pl.ds(t0 + b0, nbl)],
                                dst_v.at[pl.ds(0, nbl)])
                nb = min(nbuf, nbl)
                for i in range(nb):
                    pltpu.async_copy(xsrc_hbm.at[src_v.at[i]], rows[i], sems[i])
                full = nbl - (nbl % nb)

                @pl.loop(0, full, step=nb)
                def _edges(j):
                    for b in range(nb):
                        pltpu.make_async_copy(
                            xsrc_hbm.at[src_v.at[j + b]], rows[b], sems[b]).wait()
                        pltpu.sync_copy(rows[b], agg_sh.at[dst_v.at[j + b]],
                                        add=True)

                        @pl.when(j + b + nb < nbl)
                        def _refill(b=b):
                            pltpu.async_copy(xsrc_hbm.at[src_v.at[j + b + nb]],
                                             rows[b], sems[b])

                for r in range(full, nbl):
                    b = r % nb
                    pltpu.make_async_copy(
                        xsrc_hbm.at[src_v.at[r]], rows[b], sems[b]).wait()
                    pltpu.sync_copy(rows[b], agg_sh.at[dst_v.at[r]], add=True)

        def core_edges(k_total, core_base):
            q, r = divmod(k_total, NS)
            t0 = core_base + s * q + jnp.minimum(s, r)
            if r > 0:
                @pl.when(s < r)
                def _extra():
                    run_edges(q + 1, t0)

                if q > 0:
                    @pl.when(s >= r)
                    def _plain():
                        run_edges(q, t0)
            elif q > 0:
                run_edges(q, t0)

        @pl.when(c == 0)
        def _core0():
            core_edges(k0_total, 0)

        @pl.when(c == 1)
        def _core1():
            core_edges(k1_total, k0_total)

        plsc.subcore_barrier()

        pltpu.sync_copy(agg_sh.at[pl.ds(base, rpt)],
                        out_hbm.at[pl.ds(c * n_acc + base, rpt)])

    return seg_kernel(xsrc, src2, dst2, zeros_hbm)


def kernel(x, edge_index,
           enc1_proj_W, enc1_proj_b, enc1_lin_l_W, enc1_lin_l_b, enc1_lin_r_W,
           enc2_proj_W, enc2_proj_b, enc2_lin_l_W, enc2_lin_l_b, enc2_lin_r_W,
           dec1_proj_W, dec1_proj_b, dec1_lin_l_W, dec1_lin_l_b, dec1_lin_r_W,
           dec2_proj_W, dec2_proj_b, dec2_lin_l_W, dec2_lin_l_b, dec2_lin_r_W):
    n, d_in = x.shape
    e = edge_index.shape[1]
    n_acc = _ceil_to(n, NS)

    pad = _ceil_to(e, CHUNK) - e
    if pad:
        n_acc = _ceil_to(n + 1, NS)
        src_flat = jnp.concatenate([edge_index[0], jnp.zeros((pad,), jnp.int32)])
        dst_flat = jnp.concatenate([edge_index[1], jnp.full((pad,), n, jnp.int32)])
    else:
        src_flat = edge_index[0]
        dst_flat = edge_index[1]
    k_ch = (e + pad) // CHUNK
    src2 = src_flat.reshape(k_ch, CHUNK)
    dst2 = dst_flat.reshape(k_ch, CHUNK)

    frac0 = {128: 0.72, 64: 0.64, 32: 0.57}

    def seg(xsrc, k0_total):
        d = xsrc.shape[1]
        zeros_hbm = jnp.zeros((CHUNK, d), jnp.float32)
        return _segment_partials(xsrc, src2, dst2, zeros_hbm, n_acc, k0_total)

    def k0_for(d):
        return min(k_ch - 1, max(1, round(k_ch * frac0.get(d, 0.5))))

    xsrc = _proj_kernel(x, enc1_proj_W, enc1_proj_b.reshape(1, -1))
    parts = seg(xsrc, k0_for(d_in))
    h, xsrc = _tail_kernel(parts, x, enc1_lin_l_W, enc1_lin_l_b.reshape(1, -1),
                           enc1_lin_r_W, True, (enc2_proj_W, enc2_proj_b))
    parts = seg(xsrc, k0_for(h.shape[1]))
    z, xsrc = _tail_kernel(parts, h, enc2_lin_l_W, enc2_lin_l_b.reshape(1, -1),
                           enc2_lin_r_W, False, (dec1_proj_W, dec1_proj_b))
    parts = seg(xsrc, k0_for(z.shape[1]))
    h2, xsrc = _tail_kernel(parts, z, dec1_lin_l_W, dec1_lin_l_b.reshape(1, -1),
                            dec1_lin_r_W, True, (dec2_proj_W, dec2_proj_b))
    parts = seg(xsrc, k0_for(h2.shape[1]))
    xr = _tail_kernel(parts, h2, dec2_lin_l_W, dec2_lin_l_b.reshape(1, -1),
                      dec2_lin_r_W, False)
    return (xr, z)
